# baseline (speedup 1.0000x reference)
import functools

import jax
import jax.numpy as jnp
from jax import lax
from jax.experimental import pallas as pl
from jax.experimental.pallas import tpu as pltpu
from jax.experimental.pallas import tpu_sc as plsc

VOCAB = 1000000
EMB = 64
NC = 2
NS = 16
NW = NC * NS
B = 4096 * 200
B_PER_W = B // NW
CHUNK = 128
KK = 4
GROUP = KK * CHUNK
NGROUPS = B_PER_W // GROUP
NCHUNKS_W = B_PER_W // CHUNK


def _gather_kernel(idx_hbm, table_hbm, out_hbm, idx_v, rows_v, sem):
    c = lax.axis_index("c")
    s = lax.axis_index("s")
    wid = s * NC + c
    base = wid * B_PER_W
    pltpu.sync_copy(idx_hbm.at[wid], idx_v)

    def body(g, carry):
        handles = []
        for j in range(KK):
            h = pltpu.async_copy(
                table_hbm.at[idx_v.at[g * KK + j]],
                rows_v.at[pl.ds(j * CHUNK, CHUNK)],
                sem,
            )
            handles.append(h)
        for h in handles:
            h.wait()
        pltpu.sync_copy(rows_v, out_hbm.at[pl.ds(base + g * GROUP, GROUP)])
        return carry

    lax.fori_loop(0, NGROUPS, body, 0)


@jax.jit
def _run(idx, tvectors):
    mesh = plsc.VectorSubcoreMesh(core_axis_name="c", subcore_axis_name="s")
    k = functools.partial(
        pl.kernel,
        mesh=mesh,
        out_type=jax.ShapeDtypeStruct((B, EMB), jnp.float32),
        scratch_types=[
            pltpu.VMEM((NCHUNKS_W, CHUNK), jnp.int32),
            pltpu.VMEM((GROUP, EMB), jnp.float32),
            pltpu.SemaphoreType.DMA,
        ],
        compiler_params=pltpu.CompilerParams(use_tc_tiling_on_sc=False),
    )(_gather_kernel)
    return k(idx, tvectors)


def kernel(data, tvectors):
    idx = data.astype(jnp.int32).reshape(NW, NCHUNKS_W, CHUNK)
    out = _run(idx, tvectors)
    return out.reshape(4096, 200, EMB)


# speedup vs baseline: 1.0181x; 1.0181x over previous
import functools

import jax
import jax.numpy as jnp
from jax import lax
from jax.experimental import pallas as pl
from jax.experimental.pallas import tpu as pltpu
from jax.experimental.pallas import tpu_sc as plsc

VOCAB = 1000000
EMB = 64
NC = 2
NS = 16
NW = NC * NS
B = 4096 * 200
B_PER_W = B // NW
CHUNK = 128
KK = 4
GROUP = KK * CHUNK
NGROUPS = B_PER_W // GROUP
NCHUNKS_W = B_PER_W // CHUNK


def _gather_kernel(idx_hbm, table_hbm, out_hbm, idx_v, rows_a, rows_b, sem_a, sem_b):
    c = lax.axis_index("c")
    s = lax.axis_index("s")
    wid = s * NC + c
    base = wid * B_PER_W
    pltpu.sync_copy(idx_hbm.at[wid], idx_v)

    def fire(g, buf, sem):
        for j in range(KK):
            pltpu.async_copy(
                table_hbm.at[idx_v.at[g * KK + j]],
                buf.at[pl.ds(j * CHUNK, CHUNK)],
                sem,
            )

    def drain(buf, sem):
        pltpu.make_async_copy(out_hbm.at[pl.ds(0, GROUP)], buf, sem).wait()

    def writeback(g, buf):
        pltpu.sync_copy(buf, out_hbm.at[pl.ds(base + g * GROUP, GROUP)])

    fire(0, rows_a, sem_a)

    def body(i, carry):
        g = 2 * i
        drain(rows_a, sem_a)
        fire(g + 1, rows_b, sem_b)
        writeback(g, rows_a)
        drain(rows_b, sem_b)
        fire(g + 2, rows_a, sem_a)
        writeback(g + 1, rows_b)
        return carry

    lax.fori_loop(0, NGROUPS // 2 - 1, body, 0)

    g = NGROUPS - 2
    drain(rows_a, sem_a)
    fire(g + 1, rows_b, sem_b)
    writeback(g, rows_a)
    drain(rows_b, sem_b)
    writeback(g + 1, rows_b)


@jax.jit
def _run(idx, tvectors):
    mesh = plsc.VectorSubcoreMesh(core_axis_name="c", subcore_axis_name="s")
    k = functools.partial(
        pl.kernel,
        mesh=mesh,
        out_type=jax.ShapeDtypeStruct((B, EMB), jnp.float32),
        scratch_types=[
            pltpu.VMEM((NCHUNKS_W, CHUNK), jnp.int32),
            pltpu.VMEM((GROUP, EMB), jnp.float32),
            pltpu.VMEM((GROUP, EMB), jnp.float32),
            pltpu.SemaphoreType.DMA,
            pltpu.SemaphoreType.DMA,
        ],
        compiler_params=pltpu.CompilerParams(use_tc_tiling_on_sc=False),
    )(_gather_kernel)
    return k(idx, tvectors)


def kernel(data, tvectors):
    idx = data.astype(jnp.int32).reshape(NW, NCHUNKS_W, CHUNK)
    out = _run(idx, tvectors)
    return out.reshape(4096, 200, EMB)


# speedup vs baseline: 1.0215x; 1.0034x over previous
import functools

import jax
import jax.numpy as jnp
from jax import lax
from jax.experimental import pallas as pl
from jax.experimental.pallas import tpu as pltpu
from jax.experimental.pallas import tpu_sc as plsc

VOCAB = 1000000
EMB = 64
NC = 2
NS = 16
NW = NC * NS
B = 4096 * 200
B_PER_W = B // NW
CHUNK = 512
KK = 1
GROUP = KK * CHUNK
NGROUPS = B_PER_W // GROUP
NCHUNKS_W = B_PER_W // CHUNK


def _gather_kernel(idx_hbm, table_hbm, out_hbm, idx_v, rows_a, rows_b, sem_a, sem_b):
    c = lax.axis_index("c")
    s = lax.axis_index("s")
    wid = s * NC + c
    base = wid * B_PER_W
    pltpu.sync_copy(idx_hbm.at[wid], idx_v)

    def fire(g, buf, sem):
        for j in range(KK):
            pltpu.async_copy(
                table_hbm.at[idx_v.at[g * KK + j]],
                buf.at[pl.ds(j * CHUNK, CHUNK)],
                sem,
            )

    def drain(buf, sem):
        pltpu.make_async_copy(out_hbm.at[pl.ds(0, GROUP)], buf, sem).wait()

    def writeback(g, buf):
        pltpu.sync_copy(buf, out_hbm.at[pl.ds(base + g * GROUP, GROUP)])

    fire(0, rows_a, sem_a)

    def body(i, carry):
        g = 2 * i
        drain(rows_a, sem_a)
        fire(g + 1, rows_b, sem_b)
        writeback(g, rows_a)
        drain(rows_b, sem_b)
        fire(g + 2, rows_a, sem_a)
        writeback(g + 1, rows_b)
        return carry

    lax.fori_loop(0, NGROUPS // 2 - 1, body, 0)

    g = NGROUPS - 2
    drain(rows_a, sem_a)
    fire(g + 1, rows_b, sem_b)
    writeback(g, rows_a)
    drain(rows_b, sem_b)
    writeback(g + 1, rows_b)


@jax.jit
def _run(idx, tvectors):
    mesh = plsc.VectorSubcoreMesh(core_axis_name="c", subcore_axis_name="s")
    k = functools.partial(
        pl.kernel,
        mesh=mesh,
        out_type=jax.ShapeDtypeStruct((B, EMB), jnp.float32),
        scratch_types=[
            pltpu.VMEM((NCHUNKS_W, CHUNK), jnp.int32),
            pltpu.VMEM((GROUP, EMB), jnp.float32),
            pltpu.VMEM((GROUP, EMB), jnp.float32),
            pltpu.SemaphoreType.DMA,
            pltpu.SemaphoreType.DMA,
        ],
        compiler_params=pltpu.CompilerParams(use_tc_tiling_on_sc=False),
    )(_gather_kernel)
    return k(idx, tvectors)


def kernel(data, tvectors):
    idx = data.astype(jnp.int32).reshape(NW, NCHUNKS_W, CHUNK)
    out = _run(idx, tvectors)
    return out.reshape(4096, 200, EMB)
